# Initial kernel scaffold; baseline (speedup 1.0000x reference)
import jax
import jax.numpy as jnp
from jax.experimental import pallas as pl


def kernel(input_features, doc_ids, loss_mask, ln_g, ln_b, W_q, W_k, W_v, W_g, W_res, xl_u, xl_v, codebook):
    raise NotImplementedError("write your pallas kernel here")


# speedup vs baseline: 2.0941x; 2.0941x over previous
import functools

import jax
import jax.numpy as jnp
from jax import lax
from jax.experimental import pallas as pl
from jax.experimental.pallas import tpu as pltpu
from jax.experimental.pallas import tpu_sc as plsc

_NEG_INF = 1e30
_F32 = jnp.float32


def _ln_body(x_ref, g_ref, b_ref, o_ref):
    x = x_ref[...]
    mu = jnp.mean(x, axis=-1, keepdims=True)
    var = jnp.mean((x - mu) ** 2, axis=-1, keepdims=True)
    o_ref[...] = (x - mu) * lax.rsqrt(var + 1e-6) * g_ref[...] + b_ref[...]


def _head_ln(x):
    mu = jnp.mean(x, axis=-1, keepdims=True)
    var = jnp.mean((x - mu) ** 2, axis=-1, keepdims=True)
    return (x - mu) * lax.rsqrt(var + 1e-6)


def _proj_body(xt_ref, wq_ref, wk_ref, wv_ref, wg_ref, u_ref, vb_ref,
               qu_ref, qv_ref, kn_ref, v_ref, gate_ref, *, tau):
    xt = xt_ref[...]
    q = jnp.dot(xt, wq_ref[0], preferred_element_type=_F32)
    qn = _head_ln(q) * (1.0 / tau)
    qu_ref[0] = qn + u_ref[0]
    qv_ref[0] = qn + vb_ref[0]
    k = jnp.dot(xt, wk_ref[0], preferred_element_type=_F32)
    kn_ref[0] = _head_ln(k)
    v_ref[0] = jnp.dot(xt, wv_ref[0], preferred_element_type=_F32)
    g = jnp.dot(xt, wg_ref[0], preferred_element_type=_F32)
    gate_ref[0] = g * jax.nn.sigmoid(g)


def _vq_body(kn_ref, cb_ref, mask_ref, z_ref, loss_ref, *, n_codes):
    hh = pl.program_id(0)
    ib = pl.program_id(1)
    kk = kn_ref[0]
    c = cb_ref[0]
    csq = jnp.sum(c * c, axis=1, keepdims=True)
    dots = lax.dot_general(c, kk, (((1,), (1,)), ((), ())),
                           preferred_element_type=_F32)
    dist = csq - 2.0 * dots
    minv = jnp.min(dist, axis=0, keepdims=True)
    row = lax.broadcasted_iota(jnp.int32, dist.shape, 0)
    z = jnp.min(jnp.where(dist <= minv, row, n_codes), axis=0, keepdims=True)
    z_ref[0, 0] = z + hh * n_codes

    ksq = lax.dot_general(jnp.ones((1, kk.shape[1]), _F32), kk * kk,
                          (((1,), (1,)), ((), ())),
                          preferred_element_type=_F32)
    local = jnp.sum(mask_ref[...] * (ksq + minv), axis=1, keepdims=True)

    @pl.when((hh == 0) & (ib == 0))
    def _():
        loss_ref[...] = jnp.zeros_like(loss_ref)

    loss_ref[...] += local


def _sc_gather(table, idx):
    _, dd = table.shape
    bt = idx.shape[0]
    info = plsc.get_sparse_core_info()
    nw = info.num_cores * info.num_subcores
    b_per_w = bt // nw
    mesh = plsc.VectorSubcoreMesh(core_axis_name="c", subcore_axis_name="s")

    @functools.partial(
        pl.kernel, mesh=mesh,
        out_type=jax.ShapeDtypeStruct((bt, dd), _F32),
        compiler_params=pltpu.CompilerParams(use_tc_tiling_on_sc=False),
        scratch_types=[
            pltpu.VMEM((b_per_w,), jnp.int32),
            pltpu.VMEM((b_per_w, dd), _F32),
            pltpu.SemaphoreType.DMA,
        ],
    )
    def k(table_hbm, idx_hbm, out_hbm, idx_v, rows_v, sem):
        wid = lax.axis_index("s") * info.num_cores + lax.axis_index("c")
        base = wid * b_per_w
        pltpu.sync_copy(idx_hbm.at[pl.ds(base, b_per_w)], idx_v)
        pltpu.async_copy(table_hbm.at[idx_v], rows_v, sem).wait()
        pltpu.sync_copy(rows_v, out_hbm.at[pl.ds(base, b_per_w)])

    return k(table, idx)


def _attn_body(qu_ref, qv_ref, kh_ref, v_ref, pg_ref, wv_ref, *, tm, l):
    ib = pl.program_id(1)
    i0 = ib * tm
    w = tm + l

    qu = qu_ref[0]
    kh = kh_ref[0]
    ac = lax.dot_general(qu, kh, (((1,), (1,)), ((), ())),
                         preferred_element_type=_F32)

    t0 = l - tm * (ib + 1)
    pslice = pg_ref[pl.ds(t0, w), :]
    g = lax.dot_general(qv_ref[0], pslice, (((1,), (1,)), ((), ())),
                        preferred_element_type=_F32)
    rolled = pltpu.roll(g, l + 1, axis=1, stride=1, stride_axis=0)
    bd = rolled[:, :l]

    scores = ac + bd
    rowi = i0 + lax.broadcasted_iota(jnp.int32, (tm, l), 0)
    coli = lax.broadcasted_iota(jnp.int32, (tm, l), 1)
    scores = jnp.where(coli <= rowi, scores, -_NEG_INF)
    m = jnp.max(scores, axis=1, keepdims=True)
    e = jnp.exp(scores - m)
    p = e / jnp.sum(e, axis=1, keepdims=True)
    wv_ref[0] = jnp.dot(p, v_ref[0], preferred_element_type=_F32)


def _out_body(wv_ref, gate_ref, x_ref, wres_ref, o_ref, *, h):
    acc = x_ref[...]
    for i in range(h):
        acc = acc + jnp.dot(wv_ref[i] * gate_ref[i], wres_ref[i],
                            preferred_element_type=_F32)
    o_ref[...] = acc


def kernel(input_features, doc_ids, loss_mask, ln_g, ln_b, W_q, W_k, W_v,
           W_g, W_res, xl_u, xl_v, codebook):
    del doc_ids
    b, l, d = input_features.shape
    h, s, dk = codebook.shape
    dv = W_v.shape[1] // h
    tau = float(dk) ** 0.5
    x2 = input_features.reshape(l, d)

    wq3 = W_q.reshape(d, h, dk).transpose(1, 0, 2)
    wk3 = W_k.reshape(d, h, dk).transpose(1, 0, 2)
    wv3 = W_v.reshape(d, h, dv).transpose(1, 0, 2)
    wg3 = W_g.reshape(d, h, dv).transpose(1, 0, 2)
    wres3 = W_res.reshape(h, dv, d)
    u3 = xl_u.reshape(h, 1, dk)
    vb3 = xl_v.reshape(h, 1, dk)
    pos = jnp.arange(l - 1, -l - 1, -1, dtype=_F32)
    inv = 1.0 / (10000.0 ** (jnp.arange(0, dk, 2, dtype=_F32) / dk))
    ang = pos[:, None] * inv[None, :]
    pg = jnp.concatenate([jnp.sin(ang), jnp.cos(ang)], axis=-1)
    mask_row = loss_mask.reshape(1, l)

    tb = 512
    xt = pl.pallas_call(
        _ln_body,
        grid=(l // tb,),
        in_specs=[
            pl.BlockSpec((tb, d), lambda i: (i, 0)),
            pl.BlockSpec((1, d), lambda i: (0, 0)),
            pl.BlockSpec((1, d), lambda i: (0, 0)),
        ],
        out_specs=pl.BlockSpec((tb, d), lambda i: (i, 0)),
        out_shape=jax.ShapeDtypeStruct((l, d), _F32),
    )(x2, ln_g.reshape(1, d), ln_b.reshape(1, d))

    hblk = lambda hh, ib: (hh, 0, 0)
    qu, qv, kn, vv, gate = pl.pallas_call(
        functools.partial(_proj_body, tau=tau),
        grid=(h, l // tb),
        in_specs=[
            pl.BlockSpec((tb, d), lambda hh, ib: (ib, 0)),
            pl.BlockSpec((1, d, dk), hblk),
            pl.BlockSpec((1, d, dk), hblk),
            pl.BlockSpec((1, d, dv), hblk),
            pl.BlockSpec((1, d, dv), hblk),
            pl.BlockSpec((1, 1, dk), hblk),
            pl.BlockSpec((1, 1, dk), hblk),
        ],
        out_specs=[
            pl.BlockSpec((1, tb, dk), lambda hh, ib: (hh, ib, 0)),
            pl.BlockSpec((1, tb, dk), lambda hh, ib: (hh, ib, 0)),
            pl.BlockSpec((1, tb, dk), lambda hh, ib: (hh, ib, 0)),
            pl.BlockSpec((1, tb, dv), lambda hh, ib: (hh, ib, 0)),
            pl.BlockSpec((1, tb, dv), lambda hh, ib: (hh, ib, 0)),
        ],
        out_shape=[
            jax.ShapeDtypeStruct((h, l, dk), _F32),
            jax.ShapeDtypeStruct((h, l, dk), _F32),
            jax.ShapeDtypeStruct((h, l, dk), _F32),
            jax.ShapeDtypeStruct((h, l, dv), _F32),
            jax.ShapeDtypeStruct((h, l, dv), _F32),
        ],
    )(xt, wq3, wk3, wv3, wg3, u3, vb3)

    lb = 256
    zidx, lacc = pl.pallas_call(
        functools.partial(_vq_body, n_codes=s),
        grid=(h, l // lb),
        in_specs=[
            pl.BlockSpec((1, lb, dk), lambda hh, ib: (hh, ib, 0)),
            pl.BlockSpec((1, s, dk), lambda hh, ib: (hh, 0, 0)),
            pl.BlockSpec((1, lb), lambda hh, ib: (0, ib)),
        ],
        out_specs=[
            pl.BlockSpec((1, 1, 1, lb), lambda hh, ib: (hh, ib, 0, 0)),
            pl.BlockSpec((1, 1), lambda hh, ib: (0, 0)),
        ],
        out_shape=[
            jax.ShapeDtypeStruct((h, l // lb, 1, lb), jnp.int32),
            jax.ShapeDtypeStruct((1, 1), _F32),
        ],
    )(kn, codebook, mask_row)

    khat = _sc_gather(codebook.reshape(h * s, dk), zidx.reshape(h * l))
    khat = khat.reshape(h, l, dk)

    tm = 256
    wv = pl.pallas_call(
        functools.partial(_attn_body, tm=tm, l=l),
        grid=(h, l // tm),
        in_specs=[
            pl.BlockSpec((1, tm, dk), lambda hh, ib: (hh, ib, 0)),
            pl.BlockSpec((1, tm, dk), lambda hh, ib: (hh, ib, 0)),
            pl.BlockSpec((1, l, dk), lambda hh, ib: (hh, 0, 0)),
            pl.BlockSpec((1, l, dv), lambda hh, ib: (hh, 0, 0)),
            pl.BlockSpec((2 * l, dk), lambda hh, ib: (0, 0)),
        ],
        out_specs=pl.BlockSpec((1, tm, dv), lambda hh, ib: (hh, ib, 0)),
        out_shape=jax.ShapeDtypeStruct((h, l, dv), _F32),
    )(qu, qv, khat, vv, pg)

    out = pl.pallas_call(
        functools.partial(_out_body, h=h),
        grid=(l // tb,),
        in_specs=[
            pl.BlockSpec((h, tb, dv), lambda i: (0, i, 0)),
            pl.BlockSpec((h, tb, dv), lambda i: (0, i, 0)),
            pl.BlockSpec((tb, d), lambda i: (i, 0)),
            pl.BlockSpec((h, dv, d), lambda i: (0, 0, 0)),
        ],
        out_specs=pl.BlockSpec((tb, d), lambda i: (i, 0)),
        out_shape=jax.ShapeDtypeStruct((l, d), _F32),
    )(wv, gate, x2, wres3)

    loss = lacc[0, 0] / (b * h * l)
    return out.reshape(b, l, d), loss, loss


# speedup vs baseline: 2.2402x; 1.0698x over previous
import functools

import jax
import jax.numpy as jnp
from jax import lax
from jax.experimental import pallas as pl
from jax.experimental.pallas import tpu as pltpu
from jax.experimental.pallas import tpu_sc as plsc

_NEG_INF = 1e30
_F32 = jnp.float32
_BF16 = jnp.bfloat16


def _ln_t_body(x_ref, g_ref, b_ref, o_ref):
    x = x_ref[...]
    mu = jnp.mean(x, axis=0, keepdims=True)
    var = jnp.mean((x - mu) ** 2, axis=0, keepdims=True)
    o_ref[...] = (x - mu) * lax.rsqrt(var + 1e-6) * g_ref[...] + b_ref[...]


def _head_ln_t(x):
    mu = jnp.mean(x, axis=0, keepdims=True)
    var = jnp.mean((x - mu) ** 2, axis=0, keepdims=True)
    return (x - mu) * lax.rsqrt(var + 1e-6)


def _proj_qk_body(xt_ref, wq_ref, wk_ref, u_ref, vb_ref,
                  qu_ref, qv_ref, kn_ref, *, tau):
    xt = xt_ref[...]
    q = lax.dot_general(wq_ref[0], xt, (((1,), (0,)), ((), ())),
                        preferred_element_type=_F32)
    qn = _head_ln_t(q) * (1.0 / tau)
    qu_ref[...] = (qn + u_ref[0]).astype(_BF16)
    qv_ref[...] = (qn + vb_ref[0]).astype(_BF16)
    k = lax.dot_general(wk_ref[0], xt, (((1,), (0,)), ((), ())),
                        preferred_element_type=_F32)
    kn_ref[...] = _head_ln_t(k)


def _proj_vg_body(x_ref, g_ref, b_ref, wv_ref, wg_ref, v_ref, gate_ref, *, h):
    x = x_ref[...]
    mu = jnp.mean(x, axis=-1, keepdims=True)
    var = jnp.mean((x - mu) ** 2, axis=-1, keepdims=True)
    xt = (x - mu) * lax.rsqrt(var + 1e-6) * g_ref[...] + b_ref[...]
    v = jnp.dot(xt, wv_ref[...], preferred_element_type=_F32)
    g = jnp.dot(xt, wg_ref[...], preferred_element_type=_F32)
    gate = g * jax.nn.sigmoid(g)
    dv = v.shape[1] // h
    for i in range(h):
        v_ref[i] = v[:, i * dv:(i + 1) * dv].astype(_BF16)
        gate_ref[i] = gate[:, i * dv:(i + 1) * dv]


def _vq_body(kn_ref, cb_ref, mask_ref, z_ref, loss_ref, *, n_codes):
    hh = pl.program_id(0)
    ib = pl.program_id(1)
    kt = kn_ref[...]
    c = cb_ref[0]
    csq = jnp.sum(c * c, axis=1, keepdims=True)
    dots = lax.dot_general(c.astype(_BF16), kt.astype(_BF16),
                           (((1,), (0,)), ((), ())),
                           preferred_element_type=_F32)
    dist = csq - 2.0 * dots
    minv = jnp.min(dist, axis=0, keepdims=True)
    row = lax.broadcasted_iota(jnp.int32, dist.shape, 0)
    z = jnp.min(jnp.where(dist <= minv, row, n_codes), axis=0, keepdims=True)
    z_ref[0, 0] = z + hh * n_codes

    ksq = jnp.sum(kt * kt, axis=0, keepdims=True)
    local = jnp.sum(mask_ref[...] * (ksq + minv), axis=1, keepdims=True)

    @pl.when((hh == 0) & (ib == 0))
    def _():
        loss_ref[...] = jnp.zeros_like(loss_ref)

    loss_ref[...] += local


def _sc_gather(table, idx):
    _, dd = table.shape
    bt = idx.shape[0]
    info = plsc.get_sparse_core_info()
    nw = info.num_cores * info.num_subcores
    b_per_w = bt // nw
    mesh = plsc.VectorSubcoreMesh(core_axis_name="c", subcore_axis_name="s")

    @functools.partial(
        pl.kernel, mesh=mesh,
        out_type=jax.ShapeDtypeStruct((bt, dd), _F32),
        compiler_params=pltpu.CompilerParams(use_tc_tiling_on_sc=False),
        scratch_types=[
            pltpu.VMEM((b_per_w,), jnp.int32),
            pltpu.VMEM((b_per_w, dd), _F32),
            pltpu.SemaphoreType.DMA,
        ],
    )
    def k(table_hbm, idx_hbm, out_hbm, idx_v, rows_v, sem):
        wid = lax.axis_index("s") * info.num_cores + lax.axis_index("c")
        base = wid * b_per_w
        pltpu.sync_copy(idx_hbm.at[pl.ds(base, b_per_w)], idx_v)
        pltpu.async_copy(table_hbm.at[idx_v], rows_v, sem).wait()
        pltpu.sync_copy(rows_v, out_hbm.at[pl.ds(base, b_per_w)])

    return k(table, idx)


def _attn_body(qu_ref, qv_ref, kh_ref, v_ref, pg_ref, wv_ref, *, tm, l):
    ib = pl.program_id(1)
    i0 = ib * tm
    w = tm + l

    ac = lax.dot_general(qu_ref[...], kh_ref[0], (((0,), (0,)), ((), ())),
                         preferred_element_type=_F32)

    t0 = l - tm * (ib + 1)
    pslice = pg_ref[:, pl.ds(t0, w)]
    g = lax.dot_general(qv_ref[...], pslice, (((0,), (0,)), ((), ())),
                        preferred_element_type=_F32)
    rolled = pltpu.roll(g, l + 1, axis=1, stride=1, stride_axis=0)
    bd = rolled[:, :l]

    scores = ac + bd
    rowi = i0 + lax.broadcasted_iota(jnp.int32, (tm, l), 0)
    coli = lax.broadcasted_iota(jnp.int32, (tm, l), 1)
    scores = jnp.where(coli <= rowi, scores, -_NEG_INF)
    m = jnp.max(scores, axis=1, keepdims=True)
    e = jnp.exp(scores - m)
    p = (e / jnp.sum(e, axis=1, keepdims=True)).astype(_BF16)
    wv_ref[0] = jnp.dot(p, v_ref[0], preferred_element_type=_F32)


def _out_body(wv_ref, gate_ref, x_ref, wres_ref, o_ref, *, h):
    acc = x_ref[...]
    for i in range(h):
        acc = acc + jnp.dot(wv_ref[i] * gate_ref[i], wres_ref[i],
                            preferred_element_type=_F32)
    o_ref[...] = acc


def kernel(input_features, doc_ids, loss_mask, ln_g, ln_b, W_q, W_k, W_v,
           W_g, W_res, xl_u, xl_v, codebook):
    del doc_ids
    b, l, d = input_features.shape
    h, s, dk = codebook.shape
    dv = W_v.shape[1] // h
    tau = float(dk) ** 0.5
    x2 = input_features.reshape(l, d)
    x2t = x2.T

    wq3 = W_q.reshape(d, h, dk).transpose(1, 2, 0)
    wk3 = W_k.reshape(d, h, dk).transpose(1, 2, 0)
    wres3 = W_res.reshape(h, dv, d)
    u3 = xl_u.reshape(h, dk, 1)
    vb3 = xl_v.reshape(h, dk, 1)
    pos = jnp.arange(l - 1, -l - 1, -1, dtype=_F32)
    inv = 1.0 / (10000.0 ** (jnp.arange(0, dk, 2, dtype=_F32) / dk))
    ang = pos[:, None] * inv[None, :]
    pgt = jnp.concatenate([jnp.sin(ang), jnp.cos(ang)],
                          axis=-1).T.astype(_BF16)
    mask_row = loss_mask.reshape(1, l)

    tb = 512
    xt = pl.pallas_call(
        _ln_t_body,
        grid=(l // tb,),
        in_specs=[
            pl.BlockSpec((d, tb), lambda i: (0, i)),
            pl.BlockSpec((d, 1), lambda i: (0, 0)),
            pl.BlockSpec((d, 1), lambda i: (0, 0)),
        ],
        out_specs=pl.BlockSpec((d, tb), lambda i: (0, i)),
        out_shape=jax.ShapeDtypeStruct((d, l), _F32),
    )(x2t, ln_g.reshape(d, 1), ln_b.reshape(d, 1))

    hblk = lambda hh, ib: (hh, 0, 0)
    tblk = lambda hh, ib: (hh, ib)
    qu, qv, kn = pl.pallas_call(
        functools.partial(_proj_qk_body, tau=tau),
        grid=(h, l // tb),
        in_specs=[
            pl.BlockSpec((d, tb), lambda hh, ib: (0, ib)),
            pl.BlockSpec((1, dk, d), hblk),
            pl.BlockSpec((1, dk, d), hblk),
            pl.BlockSpec((1, dk, 1), hblk),
            pl.BlockSpec((1, dk, 1), hblk),
        ],
        out_specs=[
            pl.BlockSpec((dk, tb), tblk),
            pl.BlockSpec((dk, tb), tblk),
            pl.BlockSpec((dk, tb), tblk),
        ],
        out_shape=[
            jax.ShapeDtypeStruct((h * dk, l), _BF16),
            jax.ShapeDtypeStruct((h * dk, l), _BF16),
            jax.ShapeDtypeStruct((h * dk, l), _F32),
        ],
    )(xt, wq3, wk3, u3, vb3)

    vv, gate = pl.pallas_call(
        functools.partial(_proj_vg_body, h=h),
        grid=(l // tb,),
        in_specs=[
            pl.BlockSpec((tb, d), lambda i: (i, 0)),
            pl.BlockSpec((1, d), lambda i: (0, 0)),
            pl.BlockSpec((1, d), lambda i: (0, 0)),
            pl.BlockSpec((d, h * dv), lambda i: (0, 0)),
            pl.BlockSpec((d, h * dv), lambda i: (0, 0)),
        ],
        out_specs=[
            pl.BlockSpec((h, tb, dv), lambda i: (0, i, 0)),
            pl.BlockSpec((h, tb, dv), lambda i: (0, i, 0)),
        ],
        out_shape=[
            jax.ShapeDtypeStruct((h, l, dv), _BF16),
            jax.ShapeDtypeStruct((h, l, dv), _F32),
        ],
    )(x2, ln_g.reshape(1, d), ln_b.reshape(1, d), W_v, W_g)

    lb = 256
    zidx, lacc = pl.pallas_call(
        functools.partial(_vq_body, n_codes=s),
        grid=(h, l // lb),
        in_specs=[
            pl.BlockSpec((dk, lb), tblk),
            pl.BlockSpec((1, s, dk), hblk),
            pl.BlockSpec((1, lb), lambda hh, ib: (0, ib)),
        ],
        out_specs=[
            pl.BlockSpec((1, 1, 1, lb), lambda hh, ib: (hh, ib, 0, 0)),
            pl.BlockSpec((1, 1), lambda hh, ib: (0, 0)),
        ],
        out_shape=[
            jax.ShapeDtypeStruct((h, l // lb, 1, lb), jnp.int32),
            jax.ShapeDtypeStruct((1, 1), _F32),
        ],
    )(kn, codebook, mask_row)

    khat = _sc_gather(codebook.reshape(h * s, dk), zidx.reshape(h * l))
    khatt = khat.reshape(h, l, dk).transpose(0, 2, 1).astype(_BF16)

    tm = 256
    wv = pl.pallas_call(
        functools.partial(_attn_body, tm=tm, l=l),
        grid=(h, l // tm),
        in_specs=[
            pl.BlockSpec((dk, tm), tblk),
            pl.BlockSpec((dk, tm), tblk),
            pl.BlockSpec((1, dk, l), hblk),
            pl.BlockSpec((1, l, dv), hblk),
            pl.BlockSpec((dk, 2 * l), lambda hh, ib: (0, 0)),
        ],
        out_specs=pl.BlockSpec((1, tm, dv), lambda hh, ib: (hh, ib, 0)),
        out_shape=jax.ShapeDtypeStruct((h, l, dv), _F32),
    )(qu, qv, khatt, vv, pgt)

    out = pl.pallas_call(
        functools.partial(_out_body, h=h),
        grid=(l // tb,),
        in_specs=[
            pl.BlockSpec((h, tb, dv), lambda i: (0, i, 0)),
            pl.BlockSpec((h, tb, dv), lambda i: (0, i, 0)),
            pl.BlockSpec((tb, d), lambda i: (i, 0)),
            pl.BlockSpec((h, dv, d), lambda i: (0, 0, 0)),
        ],
        out_specs=pl.BlockSpec((tb, d), lambda i: (i, 0)),
        out_shape=jax.ShapeDtypeStruct((l, d), _F32),
    )(wv, gate, x2, wres3)

    loss = lacc[0, 0] / (b * h * l)
    return out.reshape(b, l, d), loss, loss
